# baseline (speedup 1.0000x reference)
import jax
import jax.numpy as jnp
from jax.experimental import pallas as pl

_IMG = 224
_PW = _IMG + 2
_P = 2048
_NJ = -(-(_IMG * _PW) // _P)
_LOUT = _NJ * _P
_LIN = _LOUT + _P


def _conv_block(w_ref, b_ref, xm_ref, xn_ref, o_ref):
    c = jnp.concatenate([xm_ref[0], xn_ref[0]], axis=1)
    acc = jnp.zeros(o_ref.shape[1:], jnp.float32)
    for kh in range(3):
        for kw in range(3):
            t = kh * 3 + kw
            d = kh * _PW + kw
            acc += jax.lax.dot_general(
                w_ref[t], c[:, d:d + _P],
                dimension_numbers=(((1,), (0,)), ((), ())),
                preferred_element_type=jnp.float32,
            )
    o_ref[0] = acc + b_ref[:]


def kernel(x, kernel, bias):
    batch, cin, img, _ = x.shape
    cout = kernel.shape[0]
    wt = kernel.transpose(2, 3, 0, 1).reshape(9, cout, cin)
    b2 = bias.reshape(cout, 1)
    xp = jnp.pad(x, ((0, 0), (0, 0), (1, 1), (1, 1)))
    flat = xp.reshape(batch, cin, _PW * _PW)
    flat = jnp.pad(flat, ((0, 0), (0, 0), (0, _LIN - _PW * _PW)))

    out_flat = pl.pallas_call(
        _conv_block,
        grid=(batch, _NJ),
        in_specs=[
            pl.BlockSpec((9, cout, cin), lambda b, j: (0, 0, 0)),
            pl.BlockSpec((cout, 1), lambda b, j: (0, 0)),
            pl.BlockSpec((1, cin, _P), lambda b, j: (b, 0, j)),
            pl.BlockSpec((1, cin, _P), lambda b, j: (b, 0, j + 1)),
        ],
        out_specs=pl.BlockSpec((1, cout, _P), lambda b, j: (b, 0, j)),
        out_shape=jax.ShapeDtypeStruct((batch, cout, _LOUT), jnp.float32),
    )(wt, b2, flat, flat)

    out = out_flat[:, :, : _IMG * _PW].reshape(batch, cout, _IMG, _PW)
    return out[:, :, :, :_IMG]


# speedup vs baseline: 1.0699x; 1.0699x over previous
import jax
import jax.numpy as jnp
from jax.experimental import pallas as pl

_IMG = 224
_PW = _IMG + 2
_P = 2048
_NJ = -(-(_IMG * _PW) // _P)
_LOUT = _NJ * _P
_LIN = _LOUT + _P


def _conv_block(w_ref, b_ref, xm_ref, xn_ref, o_ref):
    c = jnp.concatenate([xm_ref[0], xn_ref[0]], axis=1)
    acc = jnp.zeros(o_ref.shape[1:], jnp.float32)
    for kh in range(3):
        for kw in range(3):
            t = kh * 3 + kw
            d = kh * _PW + kw
            acc += jax.lax.dot_general(
                w_ref[t], c[:, d:d + _P],
                dimension_numbers=(((1,), (0,)), ((), ())),
                preferred_element_type=jnp.float32,
            )
    o_ref[0] = acc + b_ref[:]


def kernel(x, kernel, bias):
    batch, cin, img, _ = x.shape
    cout = kernel.shape[0]
    wt = kernel.transpose(2, 3, 0, 1).reshape(9, cout, cin).astype(jnp.bfloat16)
    b2 = bias.reshape(cout, 1)
    xp = jnp.pad(x, ((0, 0), (0, 0), (1, 1), (1, 1)))
    flat = xp.reshape(batch, cin, _PW * _PW)
    flat = jnp.pad(flat, ((0, 0), (0, 0), (0, _LIN - _PW * _PW)))
    flat = flat.astype(jnp.bfloat16)

    out_flat = pl.pallas_call(
        _conv_block,
        grid=(batch, _NJ),
        in_specs=[
            pl.BlockSpec((9, cout, cin), lambda b, j: (0, 0, 0)),
            pl.BlockSpec((cout, 1), lambda b, j: (0, 0)),
            pl.BlockSpec((1, cin, _P), lambda b, j: (b, 0, j)),
            pl.BlockSpec((1, cin, _P), lambda b, j: (b, 0, j + 1)),
        ],
        out_specs=pl.BlockSpec((1, cout, _P), lambda b, j: (b, 0, j)),
        out_shape=jax.ShapeDtypeStruct((batch, cout, _LOUT), jnp.float32),
    )(wt, b2, flat, flat)

    out = out_flat[:, :, : _IMG * _PW].reshape(batch, cout, _IMG, _PW)
    return out[:, :, :, :_IMG]


# speedup vs baseline: 1.3617x; 1.2728x over previous
import jax
import jax.numpy as jnp
from jax.experimental import pallas as pl

_IMG = 224
_NPIX = _IMG * _IMG
_P = 1792
_NB = _NPIX // _P
_HB = 256
_HPB = _P // _HB
_NHB = _NPIX // _HB


def _conv_block(w_ref, b_ref, m_ref, xl_ref, xm_ref, xr_ref, o_ref):
    i = pl.program_id(1)
    cb = jnp.concatenate(
        [xl_ref[0].astype(jnp.bfloat16),
         xm_ref[0].astype(jnp.bfloat16),
         xr_ref[0].astype(jnp.bfloat16)], axis=1)
    ones = jnp.ones((1, _P), jnp.bfloat16)
    m_left, m_right, m_top_s, m_bot_s = (m_ref[k:k + 1, :] for k in range(4))
    m_top = jnp.where(i == 0, m_top_s, ones)
    m_bot = jnp.where(i == _NB - 1, m_bot_s, ones)
    acc = jnp.zeros((o_ref.shape[1], _P), jnp.float32)
    for kh in range(3):
        for kw in range(3):
            t = kh * 3 + kw
            o = _HB + (kh - 1) * _IMG + (kw - 1)
            bt = cb[:, o:o + _P]
            m = None
            if kw == 0:
                m = m_left
            elif kw == 2:
                m = m_right
            if kh == 0:
                m = m_top if m is None else m * m_top
            elif kh == 2:
                m = m_bot if m is None else m * m_bot
            if m is not None:
                bt = bt * m
            acc += jax.lax.dot_general(
                w_ref[t], bt,
                dimension_numbers=(((1,), (0,)), ((), ())),
                preferred_element_type=jnp.float32,
            )
    o_ref[0] = acc + b_ref[:]


def kernel(x, kernel, bias):
    batch, cin, img, _ = x.shape
    cout = kernel.shape[0]
    wt = kernel.transpose(2, 3, 0, 1).reshape(9, cout, cin).astype(jnp.bfloat16)
    b2 = bias.reshape(cout, 1)
    x3 = x.reshape(batch, cin, _NPIX)

    j = jnp.arange(_P, dtype=jnp.int32)
    w_col = j % _IMG
    masks = jnp.stack([
        (w_col != 0), (w_col != _IMG - 1),
        (j >= _IMG), (j < _P - _IMG) | (j < 0),
    ]).astype(jnp.bfloat16)

    out_flat = pl.pallas_call(
        _conv_block,
        grid=(batch, _NB),
        in_specs=[
            pl.BlockSpec((9, cout, cin), lambda b, i: (0, 0, 0)),
            pl.BlockSpec((cout, 1), lambda b, i: (0, 0)),
            pl.BlockSpec((4, _P), lambda b, i: (0, 0)),
            pl.BlockSpec((1, cin, _HB),
                         lambda b, i: (b, 0, jnp.maximum(i * _HPB - 1, 0))),
            pl.BlockSpec((1, cin, _P), lambda b, i: (b, 0, i)),
            pl.BlockSpec((1, cin, _HB),
                         lambda b, i: (b, 0, jnp.minimum(i * _HPB + _HPB,
                                                         _NHB - 1))),
        ],
        out_specs=pl.BlockSpec((1, cout, _P), lambda b, i: (b, 0, i)),
        out_shape=jax.ShapeDtypeStruct((batch, cout, _NPIX), jnp.float32),
    )(wt, b2, masks, x3, x3, x3)

    return out_flat.reshape(batch, cout, img, img)


# speedup vs baseline: 1.6425x; 1.2062x over previous
import jax
import jax.numpy as jnp
from jax.experimental import pallas as pl
from jax.experimental.pallas import tpu as pltpu

_IMG = 224
_NPIX = _IMG * _IMG
_P = 1792
_NB = _NPIX // _P
_HB = 256
_HPB = _P // _HB
_NHB = _NPIX // _HB
_CW = _P + 2 * _HB


def _conv_block(w_ref, b_ref, m_ref, xl_ref, xm_ref, xr_ref, o_ref):
    i = pl.program_id(1)
    fl = jnp.where(i == 0, 0, 1).astype(jnp.bfloat16)
    fr = jnp.where(i == _NB - 1, 0, 1).astype(jnp.bfloat16)
    cb1 = jnp.concatenate(
        [xl_ref[0].astype(jnp.bfloat16) * fl,
         xm_ref[0].astype(jnp.bfloat16),
         xr_ref[0].astype(jnp.bfloat16) * fr], axis=1)
    cbs = [cb1 * m_ref[0:1, :], cb1, cb1 * m_ref[1:2, :]]
    acc = jnp.zeros((o_ref.shape[1], _P), jnp.float32)
    for kh in range(3):
        for kw in range(3):
            t = kh * 3 + kw
            o = _HB + (kh - 1) * _IMG + (kw - 1)
            acc += jax.lax.dot_general(
                w_ref[t], cbs[kw][:, o:o + _P],
                dimension_numbers=(((1,), (0,)), ((), ())),
                preferred_element_type=jnp.float32,
            )
    o_ref[0] = acc + b_ref[:]


def kernel(x, kernel, bias):
    batch, cin, img, _ = x.shape
    cout = kernel.shape[0]
    wt = kernel.transpose(2, 3, 0, 1).reshape(9, cout, cin).astype(jnp.bfloat16)
    b2 = bias.reshape(cout, 1)
    x3 = x.reshape(batch, cin, _NPIX)

    l = jnp.arange(_CW, dtype=jnp.int32)
    col = (l + 192) % _IMG
    masks = jnp.stack([(col != _IMG - 1), (col != 0)]).astype(jnp.bfloat16)

    out_flat = pl.pallas_call(
        _conv_block,
        grid=(batch, _NB),
        in_specs=[
            pl.BlockSpec((9, cout, cin), lambda b, i: (0, 0, 0)),
            pl.BlockSpec((cout, 1), lambda b, i: (0, 0)),
            pl.BlockSpec((2, _CW), lambda b, i: (0, 0)),
            pl.BlockSpec((1, cin, _HB),
                         lambda b, i: (b, 0, jnp.maximum(i * _HPB - 1, 0))),
            pl.BlockSpec((1, cin, _P), lambda b, i: (b, 0, i)),
            pl.BlockSpec((1, cin, _HB),
                         lambda b, i: (b, 0, jnp.minimum(i * _HPB + _HPB,
                                                         _NHB - 1))),
        ],
        out_specs=pl.BlockSpec((1, cout, _P), lambda b, i: (b, 0, i)),
        out_shape=jax.ShapeDtypeStruct((batch, cout, _NPIX), jnp.float32),
        compiler_params=pltpu.CompilerParams(
            dimension_semantics=("parallel", "parallel")),
    )(wt, b2, masks, x3, x3, x3)

    return out_flat.reshape(batch, cout, img, img)


# speedup vs baseline: 1.8659x; 1.1360x over previous
import jax
import jax.numpy as jnp
from jax.experimental import pallas as pl
from jax.experimental.pallas import tpu as pltpu

_IMG = 224
_NPIX = _IMG * _IMG
_P = 3584
_NB = _NPIX // _P
_HB = 256
_HPB = _P // _HB
_NHB = _NPIX // _HB
_CW = _P + 2 * _HB


def _conv_block(w_ref, b_ref, m_ref, xl_ref, xm_ref, xr_ref, o_ref):
    i = pl.program_id(1)
    fl = jnp.where(i == 0, 0, 1).astype(jnp.bfloat16)
    fr = jnp.where(i == _NB - 1, 0, 1).astype(jnp.bfloat16)
    cb1 = jnp.concatenate(
        [xl_ref[0].astype(jnp.bfloat16) * fl,
         xm_ref[0].astype(jnp.bfloat16),
         xr_ref[0].astype(jnp.bfloat16) * fr], axis=1)
    cbs = [cb1 * m_ref[0:1, :], cb1, cb1 * m_ref[1:2, :]]
    acc = jnp.zeros((o_ref.shape[1], _P), jnp.float32)
    for kh in range(3):
        for kw in range(3):
            t = kh * 3 + kw
            o = _HB + (kh - 1) * _IMG + (kw - 1)
            acc += jax.lax.dot_general(
                w_ref[t], cbs[kw][:, o:o + _P],
                dimension_numbers=(((1,), (0,)), ((), ())),
                preferred_element_type=jnp.float32,
            )
    o_ref[0] = acc + b_ref[:]


def kernel(x, kernel, bias):
    batch, cin, img, _ = x.shape
    cout = kernel.shape[0]
    wt = kernel.transpose(2, 3, 0, 1).reshape(9, cout, cin).astype(jnp.bfloat16)
    b2 = bias.reshape(cout, 1)
    x3 = x.reshape(batch, cin, _NPIX)

    l = jnp.arange(_CW, dtype=jnp.int32)
    col = (l + 192) % _IMG
    masks = jnp.stack([(col != _IMG - 1), (col != 0)]).astype(jnp.bfloat16)

    out_flat = pl.pallas_call(
        _conv_block,
        grid=(batch, _NB),
        in_specs=[
            pl.BlockSpec((9, cout, cin), lambda b, i: (0, 0, 0)),
            pl.BlockSpec((cout, 1), lambda b, i: (0, 0)),
            pl.BlockSpec((2, _CW), lambda b, i: (0, 0)),
            pl.BlockSpec((1, cin, _HB),
                         lambda b, i: (b, 0, jnp.maximum(i * _HPB - 1, 0))),
            pl.BlockSpec((1, cin, _P), lambda b, i: (b, 0, i)),
            pl.BlockSpec((1, cin, _HB),
                         lambda b, i: (b, 0, jnp.minimum(i * _HPB + _HPB,
                                                         _NHB - 1))),
        ],
        out_specs=pl.BlockSpec((1, cout, _P), lambda b, i: (b, 0, i)),
        out_shape=jax.ShapeDtypeStruct((batch, cout, _NPIX), jnp.float32),
        compiler_params=pltpu.CompilerParams(
            dimension_semantics=("parallel", "parallel")),
    )(wt, b2, masks, x3, x3, x3)

    return out_flat.reshape(batch, cout, img, img)


# speedup vs baseline: 1.9378x; 1.0385x over previous
import jax
import jax.numpy as jnp
from jax.experimental import pallas as pl
from jax.experimental.pallas import tpu as pltpu

_IMG = 224
_NPIX = _IMG * _IMG
_P = 7168
_NB = _NPIX // _P
_HB = 256
_HPB = _P // _HB
_NHB = _NPIX // _HB
_CW = _P + 2 * _HB


def _conv_block(w_ref, b_ref, m_ref, xl_ref, xm_ref, xr_ref, o_ref):
    i = pl.program_id(1)
    fl = jnp.where(i == 0, 0, 1).astype(jnp.bfloat16)
    fr = jnp.where(i == _NB - 1, 0, 1).astype(jnp.bfloat16)
    cb1 = jnp.concatenate(
        [xl_ref[0].astype(jnp.bfloat16) * fl,
         xm_ref[0].astype(jnp.bfloat16),
         xr_ref[0].astype(jnp.bfloat16) * fr], axis=1)
    cbs = [cb1 * m_ref[0:1, :], cb1, cb1 * m_ref[1:2, :]]
    acc = jnp.zeros((o_ref.shape[1], _P), jnp.float32)
    for kh in range(3):
        for kw in range(3):
            t = kh * 3 + kw
            o = _HB + (kh - 1) * _IMG + (kw - 1)
            acc += jax.lax.dot_general(
                w_ref[t], cbs[kw][:, o:o + _P],
                dimension_numbers=(((1,), (0,)), ((), ())),
                preferred_element_type=jnp.float32,
            )
    o_ref[0] = acc + b_ref[:]


def kernel(x, kernel, bias):
    batch, cin, img, _ = x.shape
    cout = kernel.shape[0]
    wt = kernel.transpose(2, 3, 0, 1).reshape(9, cout, cin).astype(jnp.bfloat16)
    b2 = bias.reshape(cout, 1)
    x3 = x.reshape(batch, cin, _NPIX)

    l = jnp.arange(_CW, dtype=jnp.int32)
    col = (l + 192) % _IMG
    masks = jnp.stack([(col != _IMG - 1), (col != 0)]).astype(jnp.bfloat16)

    out_flat = pl.pallas_call(
        _conv_block,
        grid=(batch, _NB),
        in_specs=[
            pl.BlockSpec((9, cout, cin), lambda b, i: (0, 0, 0)),
            pl.BlockSpec((cout, 1), lambda b, i: (0, 0)),
            pl.BlockSpec((2, _CW), lambda b, i: (0, 0)),
            pl.BlockSpec((1, cin, _HB),
                         lambda b, i: (b, 0, jnp.maximum(i * _HPB - 1, 0))),
            pl.BlockSpec((1, cin, _P), lambda b, i: (b, 0, i)),
            pl.BlockSpec((1, cin, _HB),
                         lambda b, i: (b, 0, jnp.minimum(i * _HPB + _HPB,
                                                         _NHB - 1))),
        ],
        out_specs=pl.BlockSpec((1, cout, _P), lambda b, i: (b, 0, i)),
        out_shape=jax.ShapeDtypeStruct((batch, cout, _NPIX), jnp.float32),
        compiler_params=pltpu.CompilerParams(
            dimension_semantics=("parallel", "parallel")),
    )(wt, b2, masks, x3, x3, x3)

    return out_flat.reshape(batch, cout, img, img)


# speedup vs baseline: 1.9626x; 1.0128x over previous
import jax
import jax.numpy as jnp
from jax.experimental import pallas as pl
from jax.experimental.pallas import tpu as pltpu

_IMG = 224
_NPIX = _IMG * _IMG
_P = 12544
_NB = _NPIX // _P
_HB = 256
_HPB = _P // _HB
_NHB = _NPIX // _HB
_CW = _P + 2 * _HB


def _conv_block(w_ref, b_ref, m_ref, xl_ref, xm_ref, xr_ref, o_ref):
    i = pl.program_id(1)
    fl = jnp.where(i == 0, 0, 1).astype(jnp.bfloat16)
    fr = jnp.where(i == _NB - 1, 0, 1).astype(jnp.bfloat16)
    cb1 = jnp.concatenate(
        [xl_ref[0].astype(jnp.bfloat16) * fl,
         xm_ref[0].astype(jnp.bfloat16),
         xr_ref[0].astype(jnp.bfloat16) * fr], axis=1)
    cbs = [cb1 * m_ref[0:1, :], cb1, cb1 * m_ref[1:2, :]]
    acc = jnp.zeros((o_ref.shape[1], _P), jnp.float32)
    for kh in range(3):
        for kw in range(3):
            t = kh * 3 + kw
            o = _HB + (kh - 1) * _IMG + (kw - 1)
            acc += jax.lax.dot_general(
                w_ref[t], cbs[kw][:, o:o + _P],
                dimension_numbers=(((1,), (0,)), ((), ())),
                preferred_element_type=jnp.float32,
            )
    o_ref[0] = acc + b_ref[:]


def kernel(x, kernel, bias):
    batch, cin, img, _ = x.shape
    cout = kernel.shape[0]
    wt = kernel.transpose(2, 3, 0, 1).reshape(9, cout, cin).astype(jnp.bfloat16)
    b2 = bias.reshape(cout, 1)
    x3 = x.reshape(batch, cin, _NPIX)

    l = jnp.arange(_CW, dtype=jnp.int32)
    col = (l + 192) % _IMG
    masks = jnp.stack([(col != _IMG - 1), (col != 0)]).astype(jnp.bfloat16)

    out_flat = pl.pallas_call(
        _conv_block,
        grid=(batch, _NB),
        in_specs=[
            pl.BlockSpec((9, cout, cin), lambda b, i: (0, 0, 0)),
            pl.BlockSpec((cout, 1), lambda b, i: (0, 0)),
            pl.BlockSpec((2, _CW), lambda b, i: (0, 0)),
            pl.BlockSpec((1, cin, _HB),
                         lambda b, i: (b, 0, jnp.maximum(i * _HPB - 1, 0))),
            pl.BlockSpec((1, cin, _P), lambda b, i: (b, 0, i)),
            pl.BlockSpec((1, cin, _HB),
                         lambda b, i: (b, 0, jnp.minimum(i * _HPB + _HPB,
                                                         _NHB - 1))),
        ],
        out_specs=pl.BlockSpec((1, cout, _P), lambda b, i: (b, 0, i)),
        out_shape=jax.ShapeDtypeStruct((batch, cout, _NPIX), jnp.float32),
        compiler_params=pltpu.CompilerParams(
            dimension_semantics=("parallel", "parallel")),
    )(wt, b2, masks, x3, x3, x3)

    return out_flat.reshape(batch, cout, img, img)
